# baseline (speedup 1.0000x reference)
import functools

import jax
import jax.numpy as jnp
from jax import lax
from jax.experimental import pallas as pl
from jax.experimental.pallas import tpu as pltpu
from jax.experimental.pallas import tpu_sc as plsc

H = W = 384
C = 384
B = H * W
NC, NS = 2, 16
NW = NC * NS
PPW = B // NW
K = 16
NCH = PPW // K
CW = 256


def _prep_body(u_ref, v_ref, i00, i10, i01, i11, w00, w10, w01, w11):
    u = u_ref[...]
    v = v_ref[...]
    gx = u * 2.0 - 1.0
    gy = -(v * 2.0 - 1.0)
    ix = ((gx + 1.0) * W - 1.0) * 0.5
    iy = ((gy + 1.0) * H - 1.0) * 0.5
    ix0 = jnp.floor(ix)
    iy0 = jnp.floor(iy)
    ix1 = ix0 + 1.0
    iy1 = iy0 + 1.0
    wx1 = ix - ix0
    wx0 = 1.0 - wx1
    wy1 = iy - iy0
    wy0 = 1.0 - wy1
    vx0 = (ix0 >= 0) & (ix0 <= W - 1)
    vx1 = (ix1 >= 0) & (ix1 <= W - 1)
    vy0 = (iy0 >= 0) & (iy0 <= H - 1)
    vy1 = (iy1 >= 0) & (iy1 <= H - 1)
    cx0 = jnp.clip(ix0, 0, W - 1).astype(jnp.int32)
    cx1 = jnp.clip(ix1, 0, W - 1).astype(jnp.int32)
    cy0 = jnp.clip(iy0, 0, H - 1).astype(jnp.int32)
    cy1 = jnp.clip(iy1, 0, H - 1).astype(jnp.int32)
    live = u != 0.0
    zero = jnp.zeros_like(u)
    i00[...] = cy0 * W + cx0
    i10[...] = cy0 * W + cx1
    i01[...] = cy1 * W + cx0
    i11[...] = cy1 * W + cx1
    w00[...] = jnp.where(vx0 & vy0 & live, wx0 * wy0, zero)
    w10[...] = jnp.where(vx1 & vy0 & live, wx1 * wy0, zero)
    w01[...] = jnp.where(vx0 & vy1 & live, wx0 * wy1, zero)
    w11[...] = jnp.where(vx1 & vy1 & live, wx1 * wy1, zero)


def _prep(u, v):
    shp = jax.ShapeDtypeStruct((H, W), jnp.int32)
    shpf = jax.ShapeDtypeStruct((H, W), jnp.float32)
    return pl.pallas_call(
        _prep_body,
        out_shape=(shp, shp, shp, shp, shpf, shpf, shpf, shpf),
    )(u, v)


def _tr_body(in_ref, out_ref):
    out_ref[...] = in_ref[...].T


def _transpose_in(tex2):
    bw = 2048
    return pl.pallas_call(
        _tr_body,
        grid=(B // bw,),
        in_specs=[pl.BlockSpec((C, bw), lambda j: (0, j))],
        out_specs=pl.BlockSpec((bw, C), lambda j: (j, 0)),
        out_shape=jax.ShapeDtypeStruct((B, C), jnp.float32),
    )(tex2)


def _transpose_out(rows):
    bw = 2048
    return pl.pallas_call(
        _tr_body,
        grid=(B // bw,),
        in_specs=[pl.BlockSpec((bw, C), lambda j: (j, 0))],
        out_specs=pl.BlockSpec((C, bw), lambda j: (0, j)),
        out_shape=jax.ShapeDtypeStruct((C, B), jnp.float32),
    )(rows)


def _unpack2(wi):
    a = lax.bitcast_convert_type(wi << 16, jnp.float32)
    b = lax.bitcast_convert_type(wi & jnp.int32(-65536), jnp.float32)
    return a, b


def _pack2(pa, pb):
    ai = lax.bitcast_convert_type(pa, jnp.int32) + jnp.int32(0x8000)
    bi = lax.bitcast_convert_type(pb, jnp.int32) + jnp.int32(0x8000)
    return ((ai >> 16) & jnp.int32(0xFFFF)) | (bi & jnp.int32(-65536))


def _sc_body(table, idxh, wgth, out_hbm,
             idxv, wgtv, r0, r1, o0, o1,
             sg0, sg1, so0, so1):
    wid = lax.axis_index("s") * NC + lax.axis_index("c")
    base = wid * PPW
    rbuf = (r0, r1)
    obuf = (o0, o1)
    sg = (sg0, sg1)
    so = (so0, so1)

    pltpu.sync_copy(idxh.at[pl.ds(base * 4, PPW * 4)], idxv)
    pltpu.sync_copy(wgth.at[pl.ds(base * 4, PPW * 4)], wgtv)

    def fire(b, g):
        off4 = pl.multiple_of(g * (4 * K), 4 * K)
        pltpu.async_copy(table.at[idxv.at[pl.ds(off4, 4 * K)]], rbuf[b], sg[b])

    def drain_gather(b):
        pltpu.make_async_copy(table.at[pl.ds(0, 4 * K)], rbuf[b], sg[b]).wait()


    def drain_out(b):
        pltpu.make_async_copy(out_hbm.at[pl.ds(0, K)], obuf[b], so[b]).wait()

    fire(0, 0)
    fire(1, 1)

    def loop2(g2, carry):
        for b in range(2):
            g = g2 * 2 + b
            off4 = pl.multiple_of(g * (4 * K), 4 * K)
            wv00 = wgtv[pl.ds(off4, 16)]
            wv10 = wgtv[pl.ds(off4 + K, 16)]
            wv01 = wgtv[pl.ds(off4 + 2 * K, 16)]
            wv11 = wgtv[pl.ds(off4 + 3 * K, 16)]
            drain_gather(b)
            with jax.named_scope("drain_out"):
                @pl.when(g2 > 0)
                def _():
                    drain_out(b)
            rb = rbuf[b]
            ob = obuf[b]

            for k in range(K):
                w00b = jnp.full((16,), wv00[k], jnp.float32)
                w10b = jnp.full((16,), wv10[k], jnp.float32)
                w01b = jnp.full((16,), wv01[k], jnp.float32)
                w11b = jnp.full((16,), wv11[k], jnp.float32)
                for j in range(C // 32):
                    sl = pl.ds(16 * j, 16)
                    a00, b00 = _unpack2(rb[k, sl])
                    a10, b10 = _unpack2(rb[k + K, sl])
                    a01, b01 = _unpack2(rb[k + 2 * K, sl])
                    a11, b11 = _unpack2(rb[k + 3 * K, sl])
                    pa = a00 * w00b + a10 * w10b + a01 * w01b + a11 * w11b
                    pb = b00 * w00b + b10 * w10b + b01 * w01b + b11 * w11b
                    ob[k, sl] = _pack2(pa, pb)
            pltpu.async_copy(ob, out_hbm.at[pl.ds(base + g * K, K)], so[b])
            with jax.named_scope("refire"):
                @pl.when(g2 < NCH // 2 - 1)
                def _():
                    fire(b, g + 2)
        return carry

    lax.fori_loop(0, NCH // 2, loop2, 0)
    drain_out(0)
    drain_out(1)


@functools.lru_cache(maxsize=1)
def _sc_gather():
  return functools.partial(
    pl.kernel,
    out_type=jax.ShapeDtypeStruct((B, C // 2), jnp.int32),
    mesh=plsc.VectorSubcoreMesh(core_axis_name="c", subcore_axis_name="s",
                                num_cores=NC, num_subcores=NS),
    scratch_types=[
        pltpu.VMEM((PPW * 4,), jnp.int32),
        pltpu.VMEM((PPW * 4,), jnp.float32),
        pltpu.VMEM((4 * K, CW), jnp.int32),
        pltpu.VMEM((4 * K, CW), jnp.int32),
        pltpu.VMEM((K, C // 2), jnp.int32),
        pltpu.VMEM((K, C // 2), jnp.int32),
        pltpu.SemaphoreType.DMA,
        pltpu.SemaphoreType.DMA,
        pltpu.SemaphoreType.DMA,
        pltpu.SemaphoreType.DMA,
    ],
  )(_sc_body)


def kernel(uv_map, neural_tex):
    u = uv_map[0, :, :, 0]
    v = uv_map[0, :, :, 1]
    i00, i10, i01, i11, w00, w10, w01, w11 = _prep(u, v)
    idx4 = jnp.stack([i00.reshape(B), i10.reshape(B),
                      i01.reshape(B), i11.reshape(B)])
    wgt4 = jnp.stack([w00.reshape(B), w10.reshape(B),
                      w01.reshape(B), w11.reshape(B)])
    idx_flat = idx4.reshape(4, B // K, K).transpose(1, 0, 2).reshape(-1)
    wgt_flat = wgt4.reshape(4, B // K, K).transpose(1, 0, 2).reshape(-1)
    table = jnp.transpose(neural_tex.reshape(C, B)).astype(jnp.bfloat16)
    table_i = jax.lax.bitcast_convert_type(
        table.reshape(B, C // 2, 2), jnp.int32)
    table_i = jnp.pad(table_i, ((0, 0), (0, CW - C // 2)))
    rows_i = _sc_gather()(table_i, idx_flat, wgt_flat)
    rows = jax.lax.bitcast_convert_type(rows_i, jnp.bfloat16).reshape(B, C)
    out = jnp.transpose(rows).astype(jnp.float32)
    return out.reshape(1, C, H, W)


# speedup vs baseline: 1.7762x; 1.7762x over previous
import functools

import jax
import jax.numpy as jnp
from jax import lax
from jax.experimental import pallas as pl
from jax.experimental.pallas import tpu as pltpu
from jax.experimental.pallas import tpu_sc as plsc

H = W = 384
C = 384
B = H * W
NC, NS = 2, 16
NW = NC * NS
PPW = B // NW
K = 16
NCH = PPW // K
HC = C // 2


def _prep_body(u_ref, v_ref, ia_ref, ib_ref, wa0_ref, wa1_ref,
               wb0_ref, wb1_ref):
    u = u_ref[...]
    v = v_ref[...]
    gx = u * 2.0 - 1.0
    gy = -(v * 2.0 - 1.0)
    ix = ((gx + 1.0) * W - 1.0) * 0.5
    iy = ((gy + 1.0) * H - 1.0) * 0.5
    ix0 = jnp.floor(ix)
    iy0 = jnp.floor(iy)
    ix1 = ix0 + 1.0
    iy1 = iy0 + 1.0
    wx1 = ix - ix0
    wx0 = 1.0 - wx1
    wy1 = iy - iy0
    wy0 = 1.0 - wy1
    one = jnp.ones_like(u)
    zero = jnp.zeros_like(u)
    vx0 = jnp.where((ix0 >= 0) & (ix0 <= W - 1), one, zero)
    vx1 = jnp.where((ix1 >= 0) & (ix1 <= W - 1), one, zero)
    vy0 = jnp.where((iy0 >= 0) & (iy0 <= H - 1), one, zero)
    vy1 = jnp.where((iy1 >= 0) & (iy1 <= H - 1), one, zero)
    live = jnp.where(u != 0.0, one, zero)
    basef = jnp.clip(ix0, 0.0, W - 2)
    ws0 = (vx0 * wx0 * jnp.where(ix0 == basef, one, zero)
           + vx1 * wx1 * jnp.where(ix1 == basef, one, zero))
    ws1 = (vx0 * wx0 * jnp.where(ix0 == basef + 1.0, one, zero)
           + vx1 * wx1 * jnp.where(ix1 == basef + 1.0, one, zero))
    wy0v = wy0 * vy0 * live
    wy1v = wy1 * vy1 * live
    cy0 = jnp.clip(iy0, 0, H - 1).astype(jnp.int32)
    cy1 = jnp.clip(iy1, 0, H - 1).astype(jnp.int32)
    basei = basef.astype(jnp.int32)
    ia_ref[...] = cy0 * W + basei
    ib_ref[...] = cy1 * W + basei
    wa0_ref[...] = ws0 * wy0v
    wa1_ref[...] = ws1 * wy0v
    wb0_ref[...] = ws0 * wy1v
    wb1_ref[...] = ws1 * wy1v


def _prep(u, v):
    shp = jax.ShapeDtypeStruct((H, W), jnp.int32)
    shpf = jax.ShapeDtypeStruct((H, W), jnp.float32)
    return pl.pallas_call(
        _prep_body,
        out_shape=(shp, shp, shpf, shpf, shpf, shpf),
    )(u, v)


_BWB = 512


def _pack_cols(m):
    t = jnp.transpose(m).astype(jnp.bfloat16)
    uu = lax.bitcast_convert_type(t, jnp.uint16)
    lo = uu[:, :HC].astype(jnp.int32)
    hi = uu[:, HC:].astype(jnp.int32)
    return lo | (hi << 16)


def _build_body(a_ref, n_ref, out_ref):
    a = a_ref[...]
    nx = n_ref[...]
    xs = jnp.concatenate([a[:, 1:], nx[:, :1]], axis=1)
    out_ref[...] = jnp.concatenate([_pack_cols(a), _pack_cols(xs)], axis=1)


def _build(tex2):
    nb = B // _BWB
    r = _BWB // 128
    return pl.pallas_call(
        _build_body,
        grid=(nb,),
        in_specs=[
            pl.BlockSpec((C, _BWB), lambda j: (0, j)),
            pl.BlockSpec((C, 128),
                         lambda j: (0, jnp.minimum((j + 1) * r, B // 128 - 1))),
        ],
        out_specs=pl.BlockSpec((_BWB, 2 * HC), lambda j: (j, 0)),
        out_shape=jax.ShapeDtypeStruct((B, 2 * HC), jnp.int32),
    )(tex2, tex2)


_BWO = 1024


def _unpack_body(in_ref, out_ref):
    t = in_ref[...]
    lo = lax.bitcast_convert_type(
        (t & jnp.int32(0xFFFF)).astype(jnp.uint16), jnp.bfloat16)
    hi = lax.bitcast_convert_type(
        (t >> 16).astype(jnp.uint16), jnp.bfloat16)
    full = jnp.concatenate([lo, hi], axis=1).astype(jnp.float32)
    out_ref[...] = jnp.transpose(full)


def _unpack_rows(rows_i):
    return pl.pallas_call(
        _unpack_body,
        grid=(B // _BWO,),
        in_specs=[pl.BlockSpec((_BWO, HC), lambda j: (j, 0))],
        out_specs=pl.BlockSpec((C, _BWO), lambda j: (0, j)),
        out_shape=jax.ShapeDtypeStruct((C, B), jnp.float32),
    )(rows_i)


def _unpack2(wi):
    a = lax.bitcast_convert_type(wi << 16, jnp.float32)
    b = lax.bitcast_convert_type(wi & jnp.int32(-65536), jnp.float32)
    return a, b


def _pack2(pa, pb):
    ai = lax.bitcast_convert_type(pa, jnp.int32) + jnp.int32(0x8000)
    bi = lax.bitcast_convert_type(pb, jnp.int32) + jnp.int32(0x8000)
    return ((ai >> 16) & jnp.int32(0xFFFF)) | (bi & jnp.int32(-65536))


def _sc_body(table, idxh, wgth, out_hbm,
             idxv, wgtv, r0, r1, o0, o1,
             sg0, sg1, so0, so1):
    wid = lax.axis_index("s") * NC + lax.axis_index("c")
    base = wid * PPW
    rbuf = (r0, r1)
    obuf = (o0, o1)
    sg = (sg0, sg1)
    so = (so0, so1)

    pltpu.sync_copy(idxh.at[pl.ds(base * 2, PPW * 2)], idxv)
    pltpu.sync_copy(wgth.at[pl.ds(base * 4, PPW * 4)], wgtv)

    def fire(b, g):
        off2 = pl.multiple_of(g * (2 * K), 2 * K)
        pltpu.async_copy(table.at[idxv.at[pl.ds(off2, 2 * K)]], rbuf[b], sg[b])

    def drain_gather(b):
        pltpu.make_async_copy(table.at[pl.ds(0, 2 * K)], rbuf[b], sg[b]).wait()

    def drain_out(b):
        pltpu.make_async_copy(out_hbm.at[pl.ds(0, K)], obuf[b], so[b]).wait()

    fire(0, 0)
    fire(1, 1)

    def loop2(g2, carry):
        for b in range(2):
            g = g2 * 2 + b
            off4 = pl.multiple_of(g * (4 * K), 4 * K)
            wva0 = wgtv[pl.ds(off4, 16)]
            wva1 = wgtv[pl.ds(off4 + K, 16)]
            wvb0 = wgtv[pl.ds(off4 + 2 * K, 16)]
            wvb1 = wgtv[pl.ds(off4 + 3 * K, 16)]
            drain_gather(b)
            @pl.when(g2 > 0)
            def _():
                drain_out(b)
            rb = rbuf[b]
            ob = obuf[b]

            for k in range(K):
                wa0 = jnp.full((16,), wva0[k], jnp.float32)
                wa1 = jnp.full((16,), wva1[k], jnp.float32)
                wb0 = jnp.full((16,), wvb0[k], jnp.float32)
                wb1 = jnp.full((16,), wvb1[k], jnp.float32)
                for j in range(HC // 16):
                    sl = pl.ds(16 * j, 16)
                    slh = pl.ds(HC + 16 * j, 16)
                    a1, b1 = _unpack2(rb[k, sl])
                    a2, b2 = _unpack2(rb[k, slh])
                    a3, b3 = _unpack2(rb[k + K, sl])
                    a4, b4 = _unpack2(rb[k + K, slh])
                    pa = a1 * wa0 + a2 * wa1 + a3 * wb0 + a4 * wb1
                    pb = b1 * wa0 + b2 * wa1 + b3 * wb0 + b4 * wb1
                    ob[k, sl] = _pack2(pa, pb)

            pltpu.async_copy(ob, out_hbm.at[pl.ds(base + g * K, K)], so[b])
            @pl.when(g2 < NCH // 2 - 1)
            def _():
                fire(b, g + 2)
        return carry

    lax.fori_loop(0, NCH // 2, loop2, 0)
    drain_out(0)
    drain_out(1)


@functools.lru_cache(maxsize=1)
def _sc_gather():
  return functools.partial(
    pl.kernel,
    out_type=jax.ShapeDtypeStruct((B, HC), jnp.int32),
    mesh=plsc.VectorSubcoreMesh(core_axis_name="c", subcore_axis_name="s",
                                num_cores=NC, num_subcores=NS),
    scratch_types=[
        pltpu.VMEM((PPW * 2,), jnp.int32),
        pltpu.VMEM((PPW * 4,), jnp.float32),
        pltpu.VMEM((2 * K, 2 * HC), jnp.int32),
        pltpu.VMEM((2 * K, 2 * HC), jnp.int32),
        pltpu.VMEM((K, HC), jnp.int32),
        pltpu.VMEM((K, HC), jnp.int32),
        pltpu.SemaphoreType.DMA,
        pltpu.SemaphoreType.DMA,
        pltpu.SemaphoreType.DMA,
        pltpu.SemaphoreType.DMA,
    ],
  )(_sc_body)


def kernel(uv_map, neural_tex):
    u = uv_map[0, :, :, 0]
    v = uv_map[0, :, :, 1]
    ia, ib, wa0, wa1, wb0, wb1 = _prep(u, v)
    idx2 = jnp.stack([ia.reshape(B), ib.reshape(B)])
    wgt4 = jnp.stack([wa0.reshape(B), wa1.reshape(B),
                      wb0.reshape(B), wb1.reshape(B)])
    idx_flat = idx2.reshape(2, B // K, K).transpose(1, 0, 2).reshape(-1)
    wgt_flat = wgt4.reshape(4, B // K, K).transpose(1, 0, 2).reshape(-1)
    table = _build(neural_tex.reshape(C, B))
    rows_i = _sc_gather()(table, idx_flat, wgt_flat)
    out = _unpack_rows(rows_i)
    return out.reshape(1, C, H, W)


# speedup vs baseline: 3.0412x; 1.7122x over previous
import functools

import jax
import jax.numpy as jnp
from jax import lax
from jax.experimental import pallas as pl
from jax.experimental.pallas import tpu as pltpu
from jax.experimental.pallas import tpu_sc as plsc

H = W = 384
C = 384
B = H * W
NC, NS = 2, 16
NW = NC * NS
PPW = B // NW
K = 16
NCH = PPW // K


def _prep_body(u_ref, v_ref, i00, i10, i01, i11, w00, w10, w01, w11):
    u = u_ref[...]
    v = v_ref[...]
    gx = u * 2.0 - 1.0
    gy = -(v * 2.0 - 1.0)
    ix = ((gx + 1.0) * W - 1.0) * 0.5
    iy = ((gy + 1.0) * H - 1.0) * 0.5
    ix0 = jnp.floor(ix)
    iy0 = jnp.floor(iy)
    ix1 = ix0 + 1.0
    iy1 = iy0 + 1.0
    wx1 = ix - ix0
    wx0 = 1.0 - wx1
    wy1 = iy - iy0
    wy0 = 1.0 - wy1
    vx0 = (ix0 >= 0) & (ix0 <= W - 1)
    vx1 = (ix1 >= 0) & (ix1 <= W - 1)
    vy0 = (iy0 >= 0) & (iy0 <= H - 1)
    vy1 = (iy1 >= 0) & (iy1 <= H - 1)
    cx0 = jnp.clip(ix0, 0, W - 1).astype(jnp.int32)
    cx1 = jnp.clip(ix1, 0, W - 1).astype(jnp.int32)
    cy0 = jnp.clip(iy0, 0, H - 1).astype(jnp.int32)
    cy1 = jnp.clip(iy1, 0, H - 1).astype(jnp.int32)
    live = u != 0.0
    zero = jnp.zeros_like(u)
    i00[...] = cy0 * W + cx0
    i10[...] = cy0 * W + cx1
    i01[...] = cy1 * W + cx0
    i11[...] = cy1 * W + cx1
    w00[...] = jnp.where(vx0 & vy0 & live, wx0 * wy0, zero)
    w10[...] = jnp.where(vx1 & vy0 & live, wx1 * wy0, zero)
    w01[...] = jnp.where(vx0 & vy1 & live, wx0 * wy1, zero)
    w11[...] = jnp.where(vx1 & vy1 & live, wx1 * wy1, zero)


def _prep(u, v):
    shp = jax.ShapeDtypeStruct((H, W), jnp.int32)
    shpf = jax.ShapeDtypeStruct((H, W), jnp.float32)
    return pl.pallas_call(
        _prep_body,
        out_shape=(shp, shp, shp, shp, shpf, shpf, shpf, shpf),
    )(u, v)


def _sc_body(table, idxh, wgth, out_hbm,
             idxv, wgtv, r0, r1, o0, o1,
             wm00, wm10, wm01, wm11,
             sg0, sg1, so0, so1):
    wid = lax.axis_index("s") * NC + lax.axis_index("c")
    base = wid * PPW
    rbuf = (r0, r1)
    obuf = (o0, o1)
    sg = (sg0, sg1)
    so = (so0, so1)

    pltpu.sync_copy(idxh.at[pl.ds(base * 4, PPW * 4)], idxv)
    pltpu.sync_copy(wgth.at[pl.ds(base * 4, PPW * 4)], wgtv)

    def fire(b, g):
        off4 = pl.multiple_of(g * (4 * K), 4 * K)
        pltpu.async_copy(table.at[idxv.at[pl.ds(off4, 4 * K)]], rbuf[b], sg[b])

    def drain_gather(b):
        pltpu.make_async_copy(table.at[pl.ds(0, 4 * K)], rbuf[b], sg[b]).wait()

    def drain_out(b):
        pltpu.make_async_copy(out_hbm.at[pl.ds(0, K)], obuf[b], so[b]).wait()

    fire(0, 0)
    fire(1, 1)

    def loop2(g2, carry):
        for b in range(2):
            g = g2 * 2 + b
            off4 = pl.multiple_of(g * (4 * K), 4 * K)
            wv00 = wgtv[pl.ds(off4, 16)]
            wv10 = wgtv[pl.ds(off4 + 16, 16)]
            wv01 = wgtv[pl.ds(off4 + 32, 16)]
            wv11 = wgtv[pl.ds(off4 + 48, 16)]
            for l in range(16):
                wm00[l, :] = jnp.full((16,), wv00[l], jnp.float32)
                wm10[l, :] = jnp.full((16,), wv10[l], jnp.float32)
                wm01[l, :] = jnp.full((16,), wv01[l], jnp.float32)
                wm11[l, :] = jnp.full((16,), wv11[l], jnp.float32)
            drain_gather(b)
            @pl.when(g2 > 0)
            def _():
                drain_out(b)
            rb = rbuf[b]
            ob = obuf[b]

            def pix(k, carry2):
                w00b = wm00[k, :]
                w10b = wm10[k, :]
                w01b = wm01[k, :]
                w11b = wm11[k, :]
                for j in range(C // 16):
                    sl = pl.ds(16 * j, 16)
                    ob[k, sl] = (rb[k, sl] * w00b + rb[k + 16, sl] * w10b
                                 + rb[k + 32, sl] * w01b + rb[k + 48, sl] * w11b)
                return carry2

            lax.fori_loop(0, K, pix, 0)
            pltpu.async_copy(ob, out_hbm.at[pl.ds(base + g * K, K)], so[b])
            @pl.when(g2 < NCH // 2 - 1)
            def _():
                fire(b, g + 2)
        return carry

    lax.fori_loop(0, NCH // 2, loop2, 0)
    drain_out(0)
    drain_out(1)


@functools.lru_cache(maxsize=1)
def _sc_gather():
  return functools.partial(
    pl.kernel,
    out_type=jax.ShapeDtypeStruct((B, C), jnp.float32),
    mesh=plsc.VectorSubcoreMesh(core_axis_name="c", subcore_axis_name="s",
                                num_cores=NC, num_subcores=NS),
    scratch_types=[
        pltpu.VMEM((PPW * 4,), jnp.int32),
        pltpu.VMEM((PPW * 4,), jnp.float32),
        pltpu.VMEM((4 * K, C), jnp.float32),
        pltpu.VMEM((4 * K, C), jnp.float32),
        pltpu.VMEM((K, C), jnp.float32),
        pltpu.VMEM((K, C), jnp.float32),
        pltpu.VMEM((16, 16), jnp.float32),
        pltpu.VMEM((16, 16), jnp.float32),
        pltpu.VMEM((16, 16), jnp.float32),
        pltpu.VMEM((16, 16), jnp.float32),
        pltpu.SemaphoreType.DMA,
        pltpu.SemaphoreType.DMA,
        pltpu.SemaphoreType.DMA,
        pltpu.SemaphoreType.DMA,
    ],
  )(_sc_body)


def kernel(uv_map, neural_tex):
    u = uv_map[0, :, :, 0]
    v = uv_map[0, :, :, 1]
    i00, i10, i01, i11, w00, w10, w01, w11 = _prep(u, v)
    idx4 = jnp.stack([i00.reshape(B), i10.reshape(B),
                      i01.reshape(B), i11.reshape(B)])
    wgt4 = jnp.stack([w00.reshape(B), w10.reshape(B),
                      w01.reshape(B), w11.reshape(B)])
    idx_flat = idx4.reshape(4, B // K, K).transpose(1, 0, 2).reshape(-1)
    wgt_flat = wgt4.reshape(4, B // K, K).transpose(1, 0, 2).reshape(-1)
    table = jnp.transpose(neural_tex.reshape(C, B))
    rows = _sc_gather()(table, idx_flat, wgt_flat)
    out = jnp.transpose(rows)
    return out.reshape(1, C, H, W)
